# Initial kernel scaffold; baseline (speedup 1.0000x reference)
import jax
import jax.numpy as jnp
from jax.experimental import pallas as pl


def kernel(tf_user, tf_item, edge_index_u2i, edge_index_i2u, params):
    raise NotImplementedError("write your pallas kernel here")


# speedup vs baseline: 24.6020x; 24.6020x over previous
import functools

import jax
import jax.numpy as jnp
from jax import lax
from jax.experimental import pallas as pl
from jax.experimental.pallas import tpu as pltpu
from jax.experimental.pallas import tpu_sc as plsc

N = 50000
NPAD = 50176
E = 800000
EP = 819200
D_IN = 128
C = 64
BLK = 512
GRID = NPAD // BLK
ROWS_PER_TILE = NPAD // 16

f32 = jnp.float32


def _full_spec(shape):
    nd = len(shape)
    return pl.BlockSpec(shape, lambda i: (0,) * nd)


def _row_spec(w):
    return pl.BlockSpec((BLK, w), lambda i: (i, 0))


def _enc_body(x_ref, w_in, b_in,
              g0, be0, w0, bb0, g1, be1, w1, bb1,
              g2, be2, w2, bb2, g3, be3, w3, bb3,
              w_out, b_out, wg, deg_ref, lo_ref, hi_ref):
    x = x_ref[...]
    h = jnp.dot(x, w_in[...], preferred_element_type=f32) + b_in[...]
    for (g, be, w, bb) in ((g0, be0, w0, bb0), (g1, be1, w1, bb1),
                           (g2, be2, w2, bb2), (g3, be3, w3, bb3)):
        mu = jnp.mean(h, axis=-1, keepdims=True)
        var = jnp.mean((h - mu) * (h - mu), axis=-1, keepdims=True)
        ln = (h - mu) * lax.rsqrt(var + 1e-5) * g[...] + be[...]
        h = h + jnp.maximum(jnp.dot(ln, w[...], preferred_element_type=f32)
                            + bb[...], 0.0)
    x64 = (jnp.dot(jnp.maximum(h, 0.0), w_out[...],
                   preferred_element_type=f32) + b_out[...])
    y = jnp.dot(x64, wg[...], preferred_element_type=f32)
    y = y * lax.rsqrt(jnp.maximum(deg_ref[...], 1.0))
    lo_ref[...] = y[:, :32]
    hi_ref[...] = y[:, 32:]


def _encode_tc(tf_pad, p, wg, deg):
    ins = [tf_pad, p["W_in"], p["b_in"].reshape(1, C)]
    for blk in p["blocks"]:
        ins += [blk["g"].reshape(1, C), blk["be"].reshape(1, C),
                blk["W"], blk["b"].reshape(1, C)]
    ins += [p["W_out"], p["b_out"].reshape(1, C), wg, deg]
    in_specs = ([_row_spec(D_IN)] + [_full_spec(a.shape) for a in ins[1:-1]]
                + [_row_spec(1)])
    return pl.pallas_call(
        _enc_body,
        grid=(GRID,),
        in_specs=in_specs,
        out_specs=[_row_spec(32), _row_spec(32)],
        out_shape=[jax.ShapeDtypeStruct((NPAD, 32), f32)] * 2,
    )(*ins)


def _post_body(lo_ref, hi_ref, degd_ref, dega_ref, b1_ref, w2_ref, o_ref):
    agg = jnp.concatenate([lo_ref[...], hi_ref[...]], axis=1)
    b = lax.rsqrt(jnp.maximum(degd_ref[...], 1.0))
    h = jnp.maximum(agg * b + b1_ref[...], 0.0)
    a = lax.rsqrt(jnp.maximum(dega_ref[...], 1.0))
    o_ref[...] = jnp.dot(h, w2_ref[...], preferred_element_type=f32) * a


def _post_tc(agg_lo, agg_hi, deg_dst, deg_src2, bias1, w2p):
    return pl.pallas_call(
        _post_body,
        grid=(GRID,),
        in_specs=[_row_spec(32), _row_spec(32), _row_spec(1), _row_spec(1),
                  _full_spec((1, C)), _full_spec((C, 8))],
        out_specs=_row_spec(8),
        out_shape=jax.ShapeDtypeStruct((NPAD, 8), f32),
    )(agg_lo, agg_hi, deg_dst, deg_src2, bias1, w2p)


def _final_body(agg_ref, deg_ref, b2_ref, o_ref):
    b = lax.rsqrt(jnp.maximum(deg_ref[...], 1.0))
    o_ref[...] = agg_ref[...] * b + b2_ref[...]


def _final_tc(agg2, deg_dst, bias2p):
    return pl.pallas_call(
        _final_body,
        grid=(GRID,),
        in_specs=[_row_spec(8), _row_spec(1), _full_spec((1, 8))],
        out_specs=_row_spec(8),
        out_shape=jax.ShapeDtypeStruct((NPAD, 8), f32),
    )(agg2, deg_dst, bias2p)


_SC_MESH = plsc.VectorSubcoreMesh(core_axis_name="c", subcore_axis_name="s")
_SC_PARAMS = pltpu.CompilerParams(use_tc_tiling_on_sc=False)

EDGES_PER_TILE = EP // 16
CH = 400
NCH = EDGES_PER_TILE // CH
CHH = 1600
NCHH = EDGES_PER_TILE // CHH


@functools.partial(
    pl.kernel,
    out_type=[jax.ShapeDtypeStruct((NPAD,), f32)] * 4,
    mesh=_SC_MESH,
    compiler_params=_SC_PARAMS,
    scratch_types=[
        pltpu.VMEM((2, CHH), jnp.int32),
        pltpu.VMEM((2, CHH), jnp.int32),
        pltpu.VMEM((2, CHH), jnp.int32),
        pltpu.VMEM((2, CHH), jnp.int32),
        pltpu.VMEM((CHH,), f32),
        pltpu.VMEM_SHARED((NPAD,), f32),
        pltpu.VMEM_SHARED((NPAD,), f32),
        pltpu.SemaphoreType.DMA,
        pltpu.SemaphoreType.DMA,
        pltpu.SemaphoreType.DMA,
        pltpu.SemaphoreType.DMA,
    ],
)
def _hist_sc(ea, eb, ones_h, zer_h,
             out_su, out_di, out_si, out_du,
             x0, x1, x2, x3, ones_v, acc_a, acc_b, m0, m1, m2, m3):
    c = lax.axis_index("c")
    s = lax.axis_index("s")
    slots = ((x0, m0), (x1, m1), (x2, m2), (x3, m3))
    pltpu.sync_copy(ones_h, ones_v)
    pltpu.sync_copy(zer_h, acc_a.at[pl.ds(s * ROWS_PER_TILE, ROWS_PER_TILE)])
    pltpu.sync_copy(zer_h, acc_b.at[pl.ds(s * ROWS_PER_TILE, ROWS_PER_TILE)])
    plsc.subcore_barrier()

    def do_array(e_hbm):
        rbase = s * NCHH
        for p in range(4):
            pltpu.async_copy(e_hbm.at[rbase + p], slots[p][0], slots[p][1])

        def body(jj, _):
            for p in range(4):
                q = jj * 4 + p
                xb, ms = slots[p]
                pltpu.make_async_copy(e_hbm.at[rbase], xb, ms).wait()
                pltpu.sync_copy(ones_v, acc_a.at[xb.at[0]], add=True)
                pltpu.sync_copy(ones_v, acc_b.at[xb.at[1]], add=True)

                @pl.when(q + 4 < NCHH)
                def _():
                    pltpu.async_copy(e_hbm.at[rbase + q + 4], xb, ms)
            return 0

        lax.fori_loop(0, NCHH // 4, body, 0)

    pl.when(c == 0)(lambda: do_array(ea))
    pl.when(c == 1)(lambda: do_array(eb))
    plsc.subcore_barrier()

    def wb(acc, out_ref):
        pltpu.sync_copy(acc.at[pl.ds(s * ROWS_PER_TILE, ROWS_PER_TILE)],
                        out_ref.at[pl.ds(s * ROWS_PER_TILE, ROWS_PER_TILE)])

    pl.when(c == 0)(lambda: wb(acc_a, out_su))
    pl.when(c == 0)(lambda: wb(acc_b, out_di))
    pl.when(c == 1)(lambda: wb(acc_a, out_si))
    pl.when(c == 1)(lambda: wb(acc_b, out_du))


def _spmm_pipeline(y_hbm, e_hbm, acc, s,
                   x0, x1, x2, x3, r0, r1, m0, m1, m2, m3, g0, g1):
    xslots = ((x0, m0), (x1, m1), (x2, m2), (x3, m3))
    rslots = ((r0, g0), (r1, g1))
    rbase = s * NCH
    for p in range(4):
        pltpu.async_copy(e_hbm.at[rbase + p], xslots[p][0], xslots[p][1])
    pltpu.make_async_copy(e_hbm.at[rbase], x0, m0).wait()
    pltpu.async_copy(y_hbm.at[x0.at[0]], r0, g0)

    def body(jj, _):
        for p in range(4):
            q = jj * 4 + p
            xb, ms = xslots[p]
            xb1, ms1 = xslots[(p + 1) % 4]
            rb, gs = rslots[p % 2]
            rb1, gs1 = rslots[(p + 1) % 2]

            @pl.when(q + 1 < NCH)
            def _():
                pltpu.make_async_copy(e_hbm.at[rbase], xb1, ms1).wait()
                pltpu.async_copy(y_hbm.at[xb1.at[0]], rb1, gs1)

            pltpu.make_async_copy(y_hbm.at[xb.at[0]], rb, gs).wait()
            pltpu.sync_copy(rb, acc.at[xb.at[1]], add=True)

            @pl.when(q + 4 < NCH)
            def _():
                pltpu.async_copy(e_hbm.at[rbase + q + 4], xb, ms)
        return 0

    lax.fori_loop(0, NCH // 4, body, 0)


def _spmm_scratch(w):
    return [
        pltpu.VMEM((2, CH), jnp.int32),
        pltpu.VMEM((2, CH), jnp.int32),
        pltpu.VMEM((2, CH), jnp.int32),
        pltpu.VMEM((2, CH), jnp.int32),
        pltpu.VMEM((CH, w), f32),
        pltpu.VMEM((CH, w), f32),
        pltpu.VMEM_SHARED((NPAD, w), f32),
        pltpu.SemaphoreType.DMA,
        pltpu.SemaphoreType.DMA,
        pltpu.SemaphoreType.DMA,
        pltpu.SemaphoreType.DMA,
        pltpu.SemaphoreType.DMA,
        pltpu.SemaphoreType.DMA,
    ]


@functools.partial(
    pl.kernel,
    out_type=[jax.ShapeDtypeStruct((NPAD, 32), f32)] * 2,
    mesh=_SC_MESH,
    compiler_params=_SC_PARAMS,
    scratch_types=_spmm_scratch(32),
)
def _spmm32_sc(y_lo, y_hi, e3, zeros_h,
               o_lo, o_hi, x0, x1, x2, x3, r0, r1, acc,
               m0, m1, m2, m3, g0, g1):
    c = lax.axis_index("c")
    s = lax.axis_index("s")
    pltpu.sync_copy(zeros_h, acc.at[pl.ds(s * ROWS_PER_TILE, ROWS_PER_TILE), :])
    plsc.subcore_barrier()
    args = (x0, x1, x2, x3, r0, r1, m0, m1, m2, m3, g0, g1)
    pl.when(c == 0)(lambda: _spmm_pipeline(y_lo, e3, acc, s, *args))
    pl.when(c == 1)(lambda: _spmm_pipeline(y_hi, e3, acc, s, *args))
    plsc.subcore_barrier()

    def wb(out_ref):
        pltpu.sync_copy(acc.at[pl.ds(s * ROWS_PER_TILE, ROWS_PER_TILE), :],
                        out_ref.at[pl.ds(s * ROWS_PER_TILE, ROWS_PER_TILE), :])

    pl.when(c == 0)(lambda: wb(o_lo))
    pl.when(c == 1)(lambda: wb(o_hi))


@functools.partial(
    pl.kernel,
    out_type=[jax.ShapeDtypeStruct((NPAD, 8), f32)] * 2,
    mesh=_SC_MESH,
    compiler_params=_SC_PARAMS,
    scratch_types=_spmm_scratch(8),
)
def _spmm8_sc(y2u, y2i, e3u, e3i, zeros_h,
              o_item, o_user, x0, x1, x2, x3, r0, r1, acc,
              m0, m1, m2, m3, g0, g1):
    c = lax.axis_index("c")
    s = lax.axis_index("s")
    pltpu.sync_copy(zeros_h, acc.at[pl.ds(s * ROWS_PER_TILE, ROWS_PER_TILE), :])
    plsc.subcore_barrier()
    args = (x0, x1, x2, x3, r0, r1, m0, m1, m2, m3, g0, g1)
    pl.when(c == 0)(lambda: _spmm_pipeline(y2u, e3u, acc, s, *args))
    pl.when(c == 1)(lambda: _spmm_pipeline(y2i, e3i, acc, s, *args))
    plsc.subcore_barrier()

    def wb(out_ref):
        pltpu.sync_copy(acc.at[pl.ds(s * ROWS_PER_TILE, ROWS_PER_TILE), :],
                        out_ref.at[pl.ds(s * ROWS_PER_TILE, ROWS_PER_TILE), :])

    pl.when(c == 0)(lambda: wb(o_item))
    pl.when(c == 1)(lambda: wb(o_user))


def _prep_edges(ei):
    pad = (50000 + (jnp.arange(EP - E, dtype=jnp.int32) % (NPAD - N))).astype(jnp.int32)
    src = jnp.concatenate([ei[0], pad])
    dst = jnp.concatenate([ei[1], pad])
    e3 = jnp.stack([src.reshape(EP // CH, CH), dst.reshape(EP // CH, CH)], axis=1)
    eh3 = jnp.stack([src.reshape(EP // CHH, CHH), dst.reshape(EP // CHH, CHH)], axis=1)
    return e3, eh3


def kernel(tf_user, tf_item, edge_index_u2i, edge_index_i2u, params):
    e3u, eh3u = _prep_edges(edge_index_u2i)
    e3i, eh3i = _prep_edges(edge_index_i2u)

    ones_h = jnp.ones((CHH,), f32)
    zer_rows = jnp.zeros((ROWS_PER_TILE,), f32)
    deg_su, deg_di, deg_si, deg_du = _hist_sc(eh3u, eh3i, ones_h, zer_rows)
    deg_su = deg_su.reshape(NPAD, 1)
    deg_di = deg_di.reshape(NPAD, 1)
    deg_si = deg_si.reshape(NPAD, 1)
    deg_du = deg_du.reshape(NPAD, 1)

    tfu = jnp.pad(tf_user, ((0, NPAD - N), (0, 0)))
    tfi = jnp.pad(tf_item, ((0, NPAD - N), (0, 0)))
    l1, l2 = params["gnn"]
    y1u_lo, y1u_hi = _encode_tc(tfu, params["enc"]["user"],
                                l1["u2i"]["W"], deg_su)
    y1i_lo, y1i_hi = _encode_tc(tfi, params["enc"]["item"],
                                l1["i2u"]["W"], deg_si)

    z32 = jnp.zeros((ROWS_PER_TILE, 32), f32)
    a1i_lo, a1i_hi = _spmm32_sc(y1u_lo, y1u_hi, e3u, z32)
    a1u_lo, a1u_hi = _spmm32_sc(y1i_lo, y1i_hi, e3i, z32)

    w2u = jnp.pad(l2["u2i"]["W"], ((0, 0), (0, 6)))
    w2i = jnp.pad(l2["i2u"]["W"], ((0, 0), (0, 6)))
    y2u = _post_tc(a1u_lo, a1u_hi, deg_du, deg_su,
                   l1["i2u"]["b"].reshape(1, C), w2u)
    y2i = _post_tc(a1i_lo, a1i_hi, deg_di, deg_si,
                   l1["u2i"]["b"].reshape(1, C), w2i)

    z8 = jnp.zeros((ROWS_PER_TILE, 8), f32)
    agg2_item, agg2_user = _spmm8_sc(y2u, y2i, e3u, e3i, z8)

    b2i = jnp.pad(l2["u2i"]["b"], (0, 6)).reshape(1, 8)
    b2u = jnp.pad(l2["i2u"]["b"], (0, 6)).reshape(1, 8)
    out_item = _final_tc(agg2_item, deg_di, b2i)
    out_user = _final_tc(agg2_user, deg_du, b2u)
    return (out_user[:N, :2], out_item[:N, :2])


# speedup vs baseline: 39.2336x; 1.5947x over previous
import functools

import jax
import jax.numpy as jnp
from jax import lax
from jax.experimental import pallas as pl
from jax.experimental.pallas import tpu as pltpu
from jax.experimental.pallas import tpu_sc as plsc

N = 50000
NPAD = 50176
E = 800000
D_IN = 128
C = 64
ROWS_PER_TILE = NPAD // 16

BLK_E = 1024
GRID_E = NPAD // BLK_E
BLK_P = 3584
GRID_P = NPAD // BLK_P

f32 = jnp.float32


def _full_spec(shape):
    nd = len(shape)
    return pl.BlockSpec(shape, lambda i: (0,) * nd)


def _row_spec(blk, w):
    return pl.BlockSpec((blk, w), lambda i: (i, 0))


def _enc_body(x_ref, w_in, b_in,
              g0, be0, w0, bb0, g1, be1, w1, bb1,
              g2, be2, w2, bb2, g3, be3, w3, bb3,
              w_out, b_out, wg, deg_ref, lo_ref, hi_ref):
    x = x_ref[...]
    h = jnp.dot(x, w_in[...], preferred_element_type=f32) + b_in[...]
    for (g, be, w, bb) in ((g0, be0, w0, bb0), (g1, be1, w1, bb1),
                           (g2, be2, w2, bb2), (g3, be3, w3, bb3)):
        mu = jnp.mean(h, axis=-1, keepdims=True)
        var = jnp.mean((h - mu) * (h - mu), axis=-1, keepdims=True)
        ln = (h - mu) * lax.rsqrt(var + 1e-5) * g[...] + be[...]
        h = h + jnp.maximum(jnp.dot(ln, w[...], preferred_element_type=f32)
                            + bb[...], 0.0)
    x64 = (jnp.dot(jnp.maximum(h, 0.0), w_out[...],
                   preferred_element_type=f32) + b_out[...])
    y = jnp.dot(x64, wg[...], preferred_element_type=f32)
    y = y * lax.rsqrt(jnp.maximum(deg_ref[...], 1.0))
    lo_ref[...] = y[:, :32]
    hi_ref[...] = y[:, 32:]


def _encode_tc(tf, p, wg, deg):
    ins = [tf, p["W_in"], p["b_in"].reshape(1, C)]
    for blk in p["blocks"]:
        ins += [blk["g"].reshape(1, C), blk["be"].reshape(1, C),
                blk["W"], blk["b"].reshape(1, C)]
    ins += [p["W_out"], p["b_out"].reshape(1, C), wg, deg]
    in_specs = ([_row_spec(BLK_E, D_IN)]
                + [_full_spec(a.shape) for a in ins[1:-1]]
                + [_row_spec(BLK_E, 1)])
    return pl.pallas_call(
        _enc_body,
        grid=(GRID_E,),
        in_specs=in_specs,
        out_specs=[_row_spec(BLK_E, 32), _row_spec(BLK_E, 32)],
        out_shape=[jax.ShapeDtypeStruct((NPAD, 32), f32)] * 2,
    )(*ins)


def _post_body(lo_ref, hi_ref, degd_ref, dega_ref, b1_ref, w2_ref, o_ref):
    agg = jnp.concatenate([lo_ref[...], hi_ref[...]], axis=1)
    b = lax.rsqrt(jnp.maximum(degd_ref[...], 1.0))
    h = jnp.maximum(agg * b + b1_ref[...], 0.0)
    a = lax.rsqrt(jnp.maximum(dega_ref[...], 1.0))
    o_ref[...] = jnp.dot(h, w2_ref[...], preferred_element_type=f32) * a


def _post_tc(agg_lo, agg_hi, deg_dst, deg_src2, bias1, w2p):
    return pl.pallas_call(
        _post_body,
        grid=(GRID_P,),
        in_specs=[_row_spec(BLK_P, 32), _row_spec(BLK_P, 32),
                  _row_spec(BLK_P, 1), _row_spec(BLK_P, 1),
                  _full_spec((1, C)), _full_spec((C, 8))],
        out_specs=_row_spec(BLK_P, 8),
        out_shape=jax.ShapeDtypeStruct((NPAD, 8), f32),
    )(agg_lo, agg_hi, deg_dst, deg_src2, bias1, w2p)


def _final_body(agg_ref, deg_ref, b2_ref, o_ref):
    b = lax.rsqrt(jnp.maximum(deg_ref[...], 1.0))
    o_ref[...] = agg_ref[...] * b + b2_ref[...]


def _final_tc(agg2, deg_dst, bias2p):
    return pl.pallas_call(
        _final_body,
        grid=(GRID_P,),
        in_specs=[_row_spec(BLK_P, 8), _row_spec(BLK_P, 1),
                  _full_spec((1, 8))],
        out_specs=_row_spec(BLK_P, 8),
        out_shape=jax.ShapeDtypeStruct((NPAD, 8), f32),
    )(agg2, deg_dst, bias2p)


_SC_MESH = plsc.VectorSubcoreMesh(core_axis_name="c", subcore_axis_name="s")
_SC_PARAMS = pltpu.CompilerParams(use_tc_tiling_on_sc=False)

EDGES_PER_TILE = E // 16
CH = 400
NCH = EDGES_PER_TILE // CH
CHH = 2000
NCHH = EDGES_PER_TILE // CHH


@functools.partial(
    pl.kernel,
    out_type=[jax.ShapeDtypeStruct((NPAD,), f32)] * 4,
    mesh=_SC_MESH,
    compiler_params=_SC_PARAMS,
    scratch_types=[
        pltpu.VMEM((CHH,), jnp.int32),
        pltpu.VMEM((CHH,), jnp.int32),
        pltpu.VMEM((CHH,), jnp.int32),
        pltpu.VMEM((CHH,), jnp.int32),
        pltpu.VMEM((CHH,), jnp.int32),
        pltpu.VMEM((CHH,), jnp.int32),
        pltpu.VMEM((CHH,), jnp.int32),
        pltpu.VMEM((CHH,), jnp.int32),
        pltpu.VMEM((CHH,), f32),
        pltpu.VMEM_SHARED((NPAD,), f32),
        pltpu.VMEM_SHARED((NPAD,), f32),
        pltpu.SemaphoreType.DMA,
        pltpu.SemaphoreType.DMA,
        pltpu.SemaphoreType.DMA,
        pltpu.SemaphoreType.DMA,
    ],
)
def _hist_sc(ea, eb, ones_h, zer_h,
             out_su, out_di, out_si, out_du,
             s0, s1, s2, s3, d0, d1, d2, d3, ones_v, acc_a, acc_b,
             m0, m1, m2, m3):
    c = lax.axis_index("c")
    s = lax.axis_index("s")
    base = s * EDGES_PER_TILE
    slots = ((s0, d0, m0), (s1, d1, m1), (s2, d2, m2), (s3, d3, m3))
    pltpu.sync_copy(ones_h, ones_v)
    pltpu.sync_copy(zer_h, acc_a.at[pl.ds(s * ROWS_PER_TILE, ROWS_PER_TILE)])
    pltpu.sync_copy(zer_h, acc_b.at[pl.ds(s * ROWS_PER_TILE, ROWS_PER_TILE)])
    plsc.subcore_barrier()

    def do_array(e_hbm):
        def load(q, slot):
            xs, xd, ms = slot
            pltpu.async_copy(e_hbm.at[0, pl.ds(base + q * CHH, CHH)], xs, ms)
            pltpu.async_copy(e_hbm.at[1, pl.ds(base + q * CHH, CHH)], xd, ms)

        def wait_scat(slot):
            xs, xd, ms = slot
            pltpu.make_async_copy(e_hbm.at[0, pl.ds(base, CHH)], xs, ms).wait()
            pltpu.make_async_copy(e_hbm.at[1, pl.ds(base, CHH)], xd, ms).wait()
            pltpu.sync_copy(ones_v, acc_a.at[xs], add=True)
            pltpu.sync_copy(ones_v, acc_b.at[xd], add=True)

        for p in range(4):
            load(p, slots[p])

        def body(jj, _):
            for p in range(4):
                q = jj * 4 + p
                wait_scat(slots[p])

                @pl.when(q + 4 < NCHH)
                def _():
                    load(q + 4, slots[p])
            return 0

        lax.fori_loop(0, NCHH // 4, body, 0)
        wait_scat(slots[0])

    pl.when(c == 0)(lambda: do_array(ea))
    pl.when(c == 1)(lambda: do_array(eb))
    plsc.subcore_barrier()

    def wb(acc, out_ref):
        pltpu.sync_copy(acc.at[pl.ds(s * ROWS_PER_TILE, ROWS_PER_TILE)],
                        out_ref.at[pl.ds(s * ROWS_PER_TILE, ROWS_PER_TILE)])

    pl.when(c == 0)(lambda: wb(acc_a, out_su))
    pl.when(c == 0)(lambda: wb(acc_b, out_di))
    pl.when(c == 1)(lambda: wb(acc_a, out_si))
    pl.when(c == 1)(lambda: wb(acc_b, out_du))


def _spmm_pipeline(y_hbm, e_hbm, acc, s,
                   xs, xd, r0, r1, ms, g0, g1):
    base = s * EDGES_PER_TILE
    xslots = tuple((xs[p], xd[p], ms[p]) for p in range(4))
    rslots = ((r0, g0), (r1, g1))

    def load(q, slot):
        a, d, m = slot
        pltpu.async_copy(e_hbm.at[0, pl.ds(base + q * CH, CH)], a, m)
        pltpu.async_copy(e_hbm.at[1, pl.ds(base + q * CH, CH)], d, m)

    def wait_idx(slot):
        a, d, m = slot
        pltpu.make_async_copy(e_hbm.at[0, pl.ds(base, CH)], a, m).wait()
        pltpu.make_async_copy(e_hbm.at[1, pl.ds(base, CH)], d, m).wait()

    for p in range(4):
        load(p, xslots[p])
    wait_idx(xslots[0])
    pltpu.async_copy(y_hbm.at[xslots[0][0]], r0, g0)

    def step(q, p, last):
        a, d, m = xslots[p]
        rb, gs = rslots[p % 2]
        rb1, gs1 = rslots[(p + 1) % 2]
        if not last:
            wait_idx(xslots[(p + 1) % 4])
            pltpu.async_copy(y_hbm.at[xslots[(p + 1) % 4][0]], rb1, gs1)
        pltpu.make_async_copy(y_hbm.at[a], rb, gs).wait()
        pltpu.sync_copy(rb, acc.at[d], add=True)

    def body(jj, _):
        for p in range(4):
            q = jj * 4 + p
            step(q, p, last=False)

            @pl.when(q + 4 < NCH)
            def _():
                load(q + 4, xslots[p])
        return 0

    lax.fori_loop(0, NCH // 4, body, 0)
    step(NCH - 1, 0, last=True)


def _spmm_scratch(w):
    return ([pltpu.VMEM((CH,), jnp.int32)] * 8
            + [pltpu.VMEM((CH, w), f32)] * 2
            + [pltpu.VMEM_SHARED((NPAD, w), f32)]
            + [pltpu.SemaphoreType.DMA] * 6)


@functools.partial(
    pl.kernel,
    out_type=[jax.ShapeDtypeStruct((NPAD, 32), f32)] * 2,
    mesh=_SC_MESH,
    compiler_params=_SC_PARAMS,
    scratch_types=_spmm_scratch(32),
)
def _spmm32_sc(y_lo, y_hi, e_hbm, zeros_h,
               o_lo, o_hi, xs0, xs1, xs2, xs3, xd0, xd1, xd2, xd3,
               r0, r1, acc, m0, m1, m2, m3, g0, g1):
    c = lax.axis_index("c")
    s = lax.axis_index("s")
    pltpu.sync_copy(zeros_h, acc.at[pl.ds(s * ROWS_PER_TILE, ROWS_PER_TILE), :])
    plsc.subcore_barrier()
    xs = (xs0, xs1, xs2, xs3)
    xd = (xd0, xd1, xd2, xd3)
    ms = (m0, m1, m2, m3)
    pl.when(c == 0)(lambda: _spmm_pipeline(y_lo, e_hbm, acc, s,
                                           xs, xd, r0, r1, ms, g0, g1))
    pl.when(c == 1)(lambda: _spmm_pipeline(y_hi, e_hbm, acc, s,
                                           xs, xd, r0, r1, ms, g0, g1))
    plsc.subcore_barrier()

    def wb(out_ref):
        pltpu.sync_copy(acc.at[pl.ds(s * ROWS_PER_TILE, ROWS_PER_TILE), :],
                        out_ref.at[pl.ds(s * ROWS_PER_TILE, ROWS_PER_TILE), :])

    pl.when(c == 0)(lambda: wb(o_lo))
    pl.when(c == 1)(lambda: wb(o_hi))


@functools.partial(
    pl.kernel,
    out_type=[jax.ShapeDtypeStruct((NPAD, 8), f32)] * 2,
    mesh=_SC_MESH,
    compiler_params=_SC_PARAMS,
    scratch_types=_spmm_scratch(8),
)
def _spmm8_sc(y2u, y2i, eu_hbm, ei_hbm, zeros_h,
              o_item, o_user, xs0, xs1, xs2, xs3, xd0, xd1, xd2, xd3,
              r0, r1, acc, m0, m1, m2, m3, g0, g1):
    c = lax.axis_index("c")
    s = lax.axis_index("s")
    pltpu.sync_copy(zeros_h, acc.at[pl.ds(s * ROWS_PER_TILE, ROWS_PER_TILE), :])
    plsc.subcore_barrier()
    xs = (xs0, xs1, xs2, xs3)
    xd = (xd0, xd1, xd2, xd3)
    ms = (m0, m1, m2, m3)
    pl.when(c == 0)(lambda: _spmm_pipeline(y2u, eu_hbm, acc, s,
                                           xs, xd, r0, r1, ms, g0, g1))
    pl.when(c == 1)(lambda: _spmm_pipeline(y2i, ei_hbm, acc, s,
                                           xs, xd, r0, r1, ms, g0, g1))
    plsc.subcore_barrier()

    def wb(out_ref):
        pltpu.sync_copy(acc.at[pl.ds(s * ROWS_PER_TILE, ROWS_PER_TILE), :],
                        out_ref.at[pl.ds(s * ROWS_PER_TILE, ROWS_PER_TILE), :])

    pl.when(c == 0)(lambda: wb(o_item))
    pl.when(c == 1)(lambda: wb(o_user))


def kernel(tf_user, tf_item, edge_index_u2i, edge_index_i2u, params):
    ones_h = jnp.ones((CHH,), f32)
    zer_rows = jnp.zeros((ROWS_PER_TILE,), f32)
    deg_su, deg_di, deg_si, deg_du = _hist_sc(edge_index_u2i, edge_index_i2u,
                                              ones_h, zer_rows)
    deg_su = deg_su.reshape(NPAD, 1)
    deg_di = deg_di.reshape(NPAD, 1)
    deg_si = deg_si.reshape(NPAD, 1)
    deg_du = deg_du.reshape(NPAD, 1)

    l1, l2 = params["gnn"]
    y1u_lo, y1u_hi = _encode_tc(tf_user, params["enc"]["user"],
                                l1["u2i"]["W"], deg_su)
    y1i_lo, y1i_hi = _encode_tc(tf_item, params["enc"]["item"],
                                l1["i2u"]["W"], deg_si)

    z32 = jnp.zeros((ROWS_PER_TILE, 32), f32)
    a1i_lo, a1i_hi = _spmm32_sc(y1u_lo, y1u_hi, edge_index_u2i, z32)
    a1u_lo, a1u_hi = _spmm32_sc(y1i_lo, y1i_hi, edge_index_i2u, z32)

    w2u = jnp.pad(l2["u2i"]["W"], ((0, 0), (0, 6)))
    w2i = jnp.pad(l2["i2u"]["W"], ((0, 0), (0, 6)))
    y2u = _post_tc(a1u_lo, a1u_hi, deg_du, deg_su,
                   l1["i2u"]["b"].reshape(1, C), w2u)
    y2i = _post_tc(a1i_lo, a1i_hi, deg_di, deg_si,
                   l1["u2i"]["b"].reshape(1, C), w2i)

    z8 = jnp.zeros((ROWS_PER_TILE, 8), f32)
    agg2_item, agg2_user = _spmm8_sc(y2u, y2i, edge_index_u2i,
                                     edge_index_i2u, z8)

    b2i = jnp.pad(l2["u2i"]["b"], (0, 6)).reshape(1, 8)
    b2u = jnp.pad(l2["i2u"]["b"], (0, 6)).reshape(1, 8)
    out_item = _final_tc(agg2_item, deg_di, b2i)
    out_user = _final_tc(agg2_user, deg_du, b2u)
    return (out_user[:N, :2], out_item[:N, :2])
